# baseline (speedup 1.0000x reference)
import jax
import jax.numpy as jnp
import numpy as np
from jax.experimental import pallas as pl
from jax.experimental.pallas import tpu as pltpu

B = 8
N = 128
V = 20
F = 128
M = 128
NB = 3
RV = 8
RS = 128

R = 32
T = N // R

_INV_N1 = 1.0 / (N - 1)
_INV_SQRT_N1 = float(1.0 / np.sqrt(N - 1.0))


def _silu(x):
    return x * jax.nn.sigmoid(x)


def _dot(a, b):
    return jnp.dot(a, b, preferred_element_type=jnp.float32)


def _body(vec_ref, We1_ref, be1_ref, We2_ref, be2_ref, Wx1_ref, bx1_ref,
          Wx2_ref, bx2_ref, Wxo_ref, bxo_ref, Winf_ref, binf_ref,
          Wh1_ref, bh1_ref, Wh2_ref, bh2_ref, Who_ref, bho_ref,
          Wv_ref, Ws_ref, bs_ref, vr_ref, sc_ref, vec_s, h_s):
    b = pl.program_id(0)
    bi = pl.program_id(1)
    ti = pl.program_id(2)
    i0 = ti * R
    par = jax.lax.rem(b, 2)
    prev = 1 - par

    def from_input():
        return (vec_ref[0, 0], vec_ref[0, 1], vec_ref[0, 2],
                jnp.zeros((N, F), jnp.float32),
                jnp.zeros((R, F), jnp.float32),
                vec_ref[0, 0, pl.ds(i0, R), :],
                vec_ref[0, 1, pl.ds(i0, R), :],
                vec_ref[0, 2, pl.ds(i0, R), :])

    def from_scratch():
        return (vec_s[prev, bi, 0], vec_s[prev, bi, 1], vec_s[prev, bi, 2],
                h_s[prev, bi],
                h_s[prev, bi, pl.ds(i0, R), :],
                vec_s[prev, bi, 0, pl.ds(i0, R), :],
                vec_s[prev, bi, 1, pl.ds(i0, R), :],
                vec_s[prev, bi, 2, pl.ds(i0, R), :])

    vx, vy, vz, h_all, h_r, vrx, vry, vrz = jax.lax.cond(
        b == 0, from_input, from_scratch)

    vx4 = jnp.concatenate([vx, vx, vx, vx], axis=1)
    vy4 = jnp.concatenate([vy, vy, vy, vy], axis=1)
    vz4 = jnp.concatenate([vz, vz, vz, vz], axis=1)
    vrx4 = jnp.concatenate([vrx, vrx, vrx, vrx], axis=1)
    vry4 = jnp.concatenate([vry, vry, vry, vry], axis=1)
    vrz4 = jnp.concatenate([vrz, vrz, vrz, vrz], axis=1)
    dx = vrx4[:, None, :] - vx4[None, :, :]
    dy = vry4[:, None, :] - vy4[None, :, :]
    dz = vrz4[:, None, :] - vz4[None, :, :]
    len2_4 = dx * dx + dy * dy + dz * dz + 1e-20
    length4 = jnp.sqrt(len2_4)
    len2 = len2_4[:, :, :V]

    hs_rep = jnp.broadcast_to(h_all[None, :, :], (R, N, F)).reshape(R * N, F)
    hr_rep = jnp.broadcast_to(h_r[:, None, :], (R, N, F)).reshape(R * N, F)
    ef = jnp.concatenate([len2.reshape(R * N, V), hs_rep, hr_rep], axis=1)
    m1 = _silu(_dot(ef, We1_ref[0]) + be1_ref[0])
    m_ij = _silu(_dot(m1, We2_ref[0]) + be2_ref[0])

    p = _silu(_dot(m_ij, Wx1_ref[0]) + bx1_ref[0])
    p = _silu(_dot(p, Wx2_ref[0]) + bx2_ref[0])
    px = _dot(p, Wxo_ref[0]) + bxo_ref[0]
    einf = jax.nn.sigmoid(_dot(m_ij, Winf_ref[0]) + binf_ref[0])
    m_ij = m_ij.reshape(R, N, M)

    rows = i0 + jax.lax.broadcasted_iota(jnp.int32, (R, N), 0)
    cols = jax.lax.broadcasted_iota(jnp.int32, (R, N), 1)
    mask = (rows != cols).astype(jnp.float32)

    w4 = px.reshape(R, N, 4 * V) / (1.0 + length4) * mask[:, :, None]
    mult = jnp.concatenate(
        [jnp.ones_like(vx), vx, vy, vz], axis=1)
    red = jnp.sum(w4 * mult[None, :, :], axis=1)
    wsum = red[:, 0 * V:1 * V]
    t2x = red[:, 1 * V:2 * V]
    t2y = red[:, 2 * V:3 * V]
    t2z = red[:, 3 * V:4 * V]
    nvx = vrx + (vrx * wsum - t2x) * _INV_N1
    nvy = vry + (vry * wsum - t2y) * _INV_N1
    nvz = vrz + (vrz * wsum - t2z) * _INV_N1

    em = einf.reshape(R, N, 1) * mask[:, :, None]
    m_i = jnp.sum(m_ij * em, axis=1) * _INV_SQRT_N1

    hcat = jnp.concatenate([m_i, h_r], axis=1)
    q = _silu(_dot(hcat, Wh1_ref[0]) + bh1_ref[0])
    q = _silu(_dot(q, Wh2_ref[0]) + bh2_ref[0])
    h_new = _dot(q, Who_ref[0]) + bho_ref[0] + h_r

    @pl.when(b < NB - 1)
    def _store_state():
        vec_s[par, bi, 0, pl.ds(i0, R), :] = nvx
        vec_s[par, bi, 1, pl.ds(i0, R), :] = nvy
        vec_s[par, bi, 2, pl.ds(i0, R), :] = nvz
        h_s[par, bi, pl.ds(i0, R), :] = h_new

    @pl.when(b == NB - 1)
    def _store_out():
        z = h_new - jnp.max(h_new, axis=1, keepdims=True)
        ez = jnp.exp(z)
        sm = ez / jnp.sum(ez, axis=1, keepdims=True)
        sc_ref[0] = _dot(sm, Ws_ref[...]) + bs_ref[...]
        Wv = Wv_ref[...]
        vr_ref[0, 0] = _dot(nvx, Wv)
        vr_ref[0, 1] = _dot(nvy, Wv)
        vr_ref[0, 2] = _dot(nvz, Wv)


def _blk(shape):
    nd = len(shape)
    return pl.BlockSpec((1,) + shape, lambda b, bi, ti, _n=nd: (b,) + (0,) * _n)


def _const(shape):
    nd = len(shape)
    return pl.BlockSpec(shape, lambda b, bi, ti, _n=nd: (0,) * _n)


_CALL = pl.pallas_call(
    _body,
    grid=(NB, B, T),
    in_specs=[
        pl.BlockSpec((1, 3, N, V), lambda b, bi, ti: (bi, 0, 0, 0)),
        _blk((V + 2 * F, M)), _blk((1, M)),
        _blk((M, M)), _blk((1, M)),
        _blk((M, M)), _blk((1, M)),
        _blk((M, M)), _blk((1, M)),
        _blk((M, 4 * V)), _blk((1, 4 * V)),
        _blk((M, 1)), _blk((1, 1)),
        _blk((M + F, M)), _blk((1, M)),
        _blk((M, M)), _blk((1, M)),
        _blk((M, F)), _blk((1, F)),
        _const((V, RV)), _const((F, RS)), _const((1, RS)),
    ],
    out_specs=[
        pl.BlockSpec((1, 3, R, RV), lambda b, bi, ti: (bi, 0, ti, 0)),
        pl.BlockSpec((1, R, RS), lambda b, bi, ti: (bi, ti, 0)),
    ],
    out_shape=[
        jax.ShapeDtypeStruct((B, 3, N, RV), jnp.float32),
        jax.ShapeDtypeStruct((B, N, RS), jnp.float32),
    ],
    scratch_shapes=[
        pltpu.VMEM((2, B, 3, N, V), jnp.float32),
        pltpu.VMEM((2, B, N, F), jnp.float32),
    ],
)


def kernel(x, We1, be1, We2, be2, Wx1, bx1, Wx2, bx2, Wxo, bxo, Winf, binf,
           Wh1, bh1, Wh2, bh2, Who, bho, Wv, Ws, bs):
    vec0 = x - jnp.mean(x, axis=1, keepdims=True)
    vec = jnp.broadcast_to(
        jnp.transpose(vec0, (0, 2, 1))[:, :, :, None], (B, 3, N, V))

    vr, sc = _CALL(
        vec,
        We1, be1[:, None], We2, be2[:, None], Wx1, bx1[:, None],
        Wx2, bx2[:, None], jnp.tile(Wxo, (1, 1, 4)),
        jnp.tile(bxo[:, None], (1, 1, 4)), Winf, binf[:, None],
        Wh1, bh1[:, None], Wh2, bh2[:, None], Who, bho[:, None],
        Wv, Ws, bs[None])

    vec_read = jnp.transpose(vr, (0, 2, 3, 1))
    return vec_read, sc


# speedup vs baseline: 1.0774x; 1.0774x over previous
import jax
import jax.numpy as jnp
import numpy as np
from jax.experimental import pallas as pl
from jax.experimental.pallas import tpu as pltpu

B = 8
N = 128
V = 20
F = 128
M = 128
NB = 3
RV = 8
RS = 128

R = 64
T = N // R

_INV_N1 = 1.0 / (N - 1)
_INV_SQRT_N1 = float(1.0 / np.sqrt(N - 1.0))


def _silu(x):
    return x * jax.nn.sigmoid(x)


def _dot(a, b):
    return jnp.dot(a, b, preferred_element_type=jnp.float32)


def _body(vec_ref, We1_ref, be1_ref, We2_ref, be2_ref, Wx1_ref, bx1_ref,
          Wx2_ref, bx2_ref, Wxo_ref, bxo_ref, Winf_ref, binf_ref,
          Wh1_ref, bh1_ref, Wh2_ref, bh2_ref, Who_ref, bho_ref,
          Wv_ref, Ws_ref, bs_ref, vr_ref, sc_ref, vec_s, h_s):
    b = pl.program_id(0)
    bi = pl.program_id(1)
    ti = pl.program_id(2)
    i0 = ti * R
    par = jax.lax.rem(b, 2)
    prev = 1 - par

    def from_input():
        return (vec_ref[0, 0], vec_ref[0, 1], vec_ref[0, 2],
                jnp.zeros((N, F), jnp.float32),
                jnp.zeros((R, F), jnp.float32),
                vec_ref[0, 0, pl.ds(i0, R), :],
                vec_ref[0, 1, pl.ds(i0, R), :],
                vec_ref[0, 2, pl.ds(i0, R), :])

    def from_scratch():
        return (vec_s[prev, bi, 0], vec_s[prev, bi, 1], vec_s[prev, bi, 2],
                h_s[prev, bi],
                h_s[prev, bi, pl.ds(i0, R), :],
                vec_s[prev, bi, 0, pl.ds(i0, R), :],
                vec_s[prev, bi, 1, pl.ds(i0, R), :],
                vec_s[prev, bi, 2, pl.ds(i0, R), :])

    vx, vy, vz, h_all, h_r, vrx, vry, vrz = jax.lax.cond(
        b == 0, from_input, from_scratch)

    vx4 = jnp.concatenate([vx, vx, vx, vx], axis=1)
    vy4 = jnp.concatenate([vy, vy, vy, vy], axis=1)
    vz4 = jnp.concatenate([vz, vz, vz, vz], axis=1)
    vrx4 = jnp.concatenate([vrx, vrx, vrx, vrx], axis=1)
    vry4 = jnp.concatenate([vry, vry, vry, vry], axis=1)
    vrz4 = jnp.concatenate([vrz, vrz, vrz, vrz], axis=1)
    dx = vrx4[:, None, :] - vx4[None, :, :]
    dy = vry4[:, None, :] - vy4[None, :, :]
    dz = vrz4[:, None, :] - vz4[None, :, :]
    len2_4 = dx * dx + dy * dy + dz * dz + 1e-20
    length4 = jnp.sqrt(len2_4)
    len2 = len2_4[:, :, :V]

    hs_rep = jnp.broadcast_to(h_all[None, :, :], (R, N, F)).reshape(R * N, F)
    hr_rep = jnp.broadcast_to(h_r[:, None, :], (R, N, F)).reshape(R * N, F)
    ef = jnp.concatenate([len2.reshape(R * N, V), hs_rep, hr_rep], axis=1)
    m1 = _silu(_dot(ef, We1_ref[0]) + be1_ref[0])
    m_ij = _silu(_dot(m1, We2_ref[0]) + be2_ref[0])

    p = _silu(_dot(m_ij, Wx1_ref[0]) + bx1_ref[0])
    p = _silu(_dot(p, Wx2_ref[0]) + bx2_ref[0])
    px = _dot(p, Wxo_ref[0]) + bxo_ref[0]
    einf = jax.nn.sigmoid(_dot(m_ij, Winf_ref[0]) + binf_ref[0])
    m_ij = m_ij.reshape(R, N, M)

    rows = i0 + jax.lax.broadcasted_iota(jnp.int32, (R, N), 0)
    cols = jax.lax.broadcasted_iota(jnp.int32, (R, N), 1)
    mask = (rows != cols).astype(jnp.float32)

    w4 = px.reshape(R, N, 4 * V) / (1.0 + length4) * mask[:, :, None]
    mult = jnp.concatenate(
        [jnp.ones_like(vx), vx, vy, vz], axis=1)
    red = jnp.sum(w4 * mult[None, :, :], axis=1)
    wsum = red[:, 0 * V:1 * V]
    t2x = red[:, 1 * V:2 * V]
    t2y = red[:, 2 * V:3 * V]
    t2z = red[:, 3 * V:4 * V]
    nvx = vrx + (vrx * wsum - t2x) * _INV_N1
    nvy = vry + (vry * wsum - t2y) * _INV_N1
    nvz = vrz + (vrz * wsum - t2z) * _INV_N1

    em = einf.reshape(R, N, 1) * mask[:, :, None]
    m_i = jnp.sum(m_ij * em, axis=1) * _INV_SQRT_N1

    hcat = jnp.concatenate([m_i, h_r], axis=1)
    q = _silu(_dot(hcat, Wh1_ref[0]) + bh1_ref[0])
    q = _silu(_dot(q, Wh2_ref[0]) + bh2_ref[0])
    h_new = _dot(q, Who_ref[0]) + bho_ref[0] + h_r

    @pl.when(b < NB - 1)
    def _store_state():
        vec_s[par, bi, 0, pl.ds(i0, R), :] = nvx
        vec_s[par, bi, 1, pl.ds(i0, R), :] = nvy
        vec_s[par, bi, 2, pl.ds(i0, R), :] = nvz
        h_s[par, bi, pl.ds(i0, R), :] = h_new

    @pl.when(b == NB - 1)
    def _store_out():
        z = h_new - jnp.max(h_new, axis=1, keepdims=True)
        ez = jnp.exp(z)
        sm = ez / jnp.sum(ez, axis=1, keepdims=True)
        sc_ref[0] = _dot(sm, Ws_ref[...]) + bs_ref[...]
        Wv = Wv_ref[...]
        vr_ref[0, 0] = _dot(nvx, Wv)
        vr_ref[0, 1] = _dot(nvy, Wv)
        vr_ref[0, 2] = _dot(nvz, Wv)


def _blk(shape):
    nd = len(shape)
    return pl.BlockSpec((1,) + shape, lambda b, bi, ti, _n=nd: (b,) + (0,) * _n)


def _const(shape):
    nd = len(shape)
    return pl.BlockSpec(shape, lambda b, bi, ti, _n=nd: (0,) * _n)


_CALL = pl.pallas_call(
    _body,
    grid=(NB, B, T),
    in_specs=[
        pl.BlockSpec((1, 3, N, V), lambda b, bi, ti: (bi, 0, 0, 0)),
        _blk((V + 2 * F, M)), _blk((1, M)),
        _blk((M, M)), _blk((1, M)),
        _blk((M, M)), _blk((1, M)),
        _blk((M, M)), _blk((1, M)),
        _blk((M, 4 * V)), _blk((1, 4 * V)),
        _blk((M, 1)), _blk((1, 1)),
        _blk((M + F, M)), _blk((1, M)),
        _blk((M, M)), _blk((1, M)),
        _blk((M, F)), _blk((1, F)),
        _const((V, RV)), _const((F, RS)), _const((1, RS)),
    ],
    out_specs=[
        pl.BlockSpec((1, 3, R, RV), lambda b, bi, ti: (bi, 0, ti, 0)),
        pl.BlockSpec((1, R, RS), lambda b, bi, ti: (bi, ti, 0)),
    ],
    out_shape=[
        jax.ShapeDtypeStruct((B, 3, N, RV), jnp.float32),
        jax.ShapeDtypeStruct((B, N, RS), jnp.float32),
    ],
    scratch_shapes=[
        pltpu.VMEM((2, B, 3, N, V), jnp.float32),
        pltpu.VMEM((2, B, N, F), jnp.float32),
    ],
)


def kernel(x, We1, be1, We2, be2, Wx1, bx1, Wx2, bx2, Wxo, bxo, Winf, binf,
           Wh1, bh1, Wh2, bh2, Who, bho, Wv, Ws, bs):
    vec0 = x - jnp.mean(x, axis=1, keepdims=True)
    vec = jnp.broadcast_to(
        jnp.transpose(vec0, (0, 2, 1))[:, :, :, None], (B, 3, N, V))

    vr, sc = _CALL(
        vec,
        We1, be1[:, None], We2, be2[:, None], Wx1, bx1[:, None],
        Wx2, bx2[:, None], jnp.tile(Wxo, (1, 1, 4)),
        jnp.tile(bxo[:, None], (1, 1, 4)), Winf, binf[:, None],
        Wh1, bh1[:, None], Wh2, bh2[:, None], Who, bho[:, None],
        Wv, Ws, bs[None])

    vec_read = jnp.transpose(vr, (0, 2, 3, 1))
    return vec_read, sc
